# Initial kernel scaffold; baseline (speedup 1.0000x reference)
import jax
import jax.numpy as jnp
from jax.experimental import pallas as pl


def kernel(xyz):
    raise NotImplementedError("write your pallas kernel here")


# speedup vs baseline: 26.7271x; 26.7271x over previous
import jax
import jax.numpy as jnp
from jax.experimental import pallas as pl
from jax.experimental.pallas import tpu as pltpu

_K = 16
_N = 1024
_BIG = 3.0e38


def _body(x_ref, xt_ref, out_ref):
    x = x_ref[0]
    xt = xt_ref[0]
    xb = x.astype(jnp.bfloat16)
    g = jax.lax.dot_general(xb, xb, (((0,), (0,)), ((), ())),
                            preferred_element_type=jnp.float32)
    eye = (jax.lax.broadcasted_iota(jnp.int32, (_N, _N), 0)
           == jax.lax.broadcasted_iota(jnp.int32, (_N, _N), 1))
    sq_row = jnp.sum(x * x, axis=0, keepdims=True)
    sq_col = jnp.sum(xt * xt, axis=1, keepdims=True)
    d2 = sq_col + sq_row - 2.0 * g

    t_row = jnp.full((1, _N), -_BIG, dtype=jnp.float32)
    for _ in range(_K):
        t_row = jnp.min(jnp.where(d2 > t_row, d2, _BIG), axis=0, keepdims=True)
    t_col = jnp.full((_N, 1), -_BIG, dtype=jnp.float32)
    for _ in range(_K):
        t_col = jnp.min(jnp.where(d2 > t_col, d2, _BIG), axis=1, keepdims=True)

    w = jnp.exp(-0.5 * d2)
    m = ((d2 <= t_row).astype(jnp.float32)
         + (d2 <= t_col).astype(jnp.float32))
    a = 0.5 * w * m
    deg = jnp.maximum(jnp.sum(a, axis=0, keepdims=True), 1e-6)
    deg_col = jnp.maximum(jnp.sum(a, axis=1, keepdims=True), 1e-6)
    r_row = jax.lax.rsqrt(deg)
    r_col = jax.lax.rsqrt(deg_col)
    lap = -(r_col * a * r_row)
    out_ref[0] = jnp.where(eye, 1.0 + lap, lap)


def kernel(xyz):
    b = xyz.shape[0]
    xt = jnp.transpose(xyz, (0, 2, 1))
    return pl.pallas_call(
        _body,
        grid=(b,),
        in_specs=[pl.BlockSpec((1, 3, _N), lambda i: (i, 0, 0)),
                  pl.BlockSpec((1, _N, 3), lambda i: (i, 0, 0))],
        out_specs=pl.BlockSpec((1, _N, _N), lambda i: (i, 0, 0)),
        out_shape=jax.ShapeDtypeStruct((b, _N, _N), jnp.float32),
    )(xyz, xt)


# speedup vs baseline: 38.2759x; 1.4321x over previous
import jax
import jax.numpy as jnp
from jax.experimental import pallas as pl
from jax.experimental.pallas import tpu as pltpu

_K = 16
_N = 1024
_BIG = 3.0e38


def _body(x_ref, out_ref):
    x = x_ref[0]
    xb = x.astype(jnp.bfloat16)
    g = jax.lax.dot_general(xb, xb, (((0,), (0,)), ((), ())),
                            preferred_element_type=jnp.float32)
    eye = (jax.lax.broadcasted_iota(jnp.int32, (_N, _N), 0)
           == jax.lax.broadcasted_iota(jnp.int32, (_N, _N), 1))
    sq_row = jnp.sum(x * x, axis=0, keepdims=True)
    sq_col = jnp.transpose(sq_row, (1, 0))
    d2 = sq_col + sq_row - 2.0 * g

    t_row = jnp.full((1, _N), -_BIG, dtype=jnp.float32)
    for _ in range(_K):
        t_row = jnp.min(jnp.where(d2 > t_row, d2, _BIG), axis=0, keepdims=True)
    t_col = jnp.transpose(t_row, (1, 0))

    w = jnp.exp(-0.5 * d2)
    m = ((d2 <= t_row).astype(jnp.float32)
         + (d2 <= t_col).astype(jnp.float32))
    a = 0.5 * w * m
    deg = jnp.maximum(jnp.sum(a, axis=0, keepdims=True), 1e-6)
    deg_col = jnp.maximum(jnp.sum(a, axis=1, keepdims=True), 1e-6)
    r_row = jax.lax.rsqrt(deg)
    r_col = jax.lax.rsqrt(deg_col)
    lap = -(r_col * a * r_row)
    out_ref[0] = jnp.where(eye, 1.0 + lap, lap)


def kernel(xyz):
    b = xyz.shape[0]
    return pl.pallas_call(
        _body,
        grid=(b,),
        in_specs=[pl.BlockSpec((1, 3, _N), lambda i: (i, 0, 0))],
        out_specs=pl.BlockSpec((1, _N, _N), lambda i: (i, 0, 0)),
        out_shape=jax.ShapeDtypeStruct((b, _N, _N), jnp.float32),
    )(xyz)
